# Initial kernel scaffold; baseline (speedup 1.0000x reference)
import jax
import jax.numpy as jnp
from jax.experimental import pallas as pl


def kernel(x, mess, nei_idx, o_idx, contexts, tree_vecs, pred_targets, stop_targets, W_z, b_z, W_r, b_r, W_h, b_h, W_ui, b_ui, W_w, b_w, W_wo, b_wo, W_u, b_u, W_uo, b_uo):
    raise NotImplementedError("write your pallas kernel here")


# speedup vs baseline: 3.2189x; 3.2189x over previous
import functools

import jax
import jax.numpy as jnp
from jax import lax
from jax.experimental import pallas as pl
from jax.experimental.pallas import tpu as pltpu
from jax.experimental.pallas import tpu_sc as plsc

HID = 128
LAT = 64
V = 6
MAXNB = 10
N = 160000
E = 160000
B = 1024

NC = 2
NS = 16
NW = NC * NS
NPW = N // NW
C = 40
NCHUNK = NPW // C
G = 2 * C
NG = C * MAXNB // G
TOTCHUNK = N // C


def _sc_gather_body(mess, nei3, o3, ctx3, tree,
                    sumh, curo, tctx,
                    idx_n, idx_o, idx_c, rows_n, rows_o, acc, trow,
                    sem_n, sem_o, sem_c):
    w = lax.axis_index("s") * NC + lax.axis_index("c")

    def accumulate(rows, out_hbm, nb):
        def per_node(i, _):
            base = i * MAXNB
            for c8 in range(HID // 16):
                sl = pl.ds(c8 * 16, 16)
                tot = rows[base, sl]
                for j in range(1, MAXNB):
                    tot = tot + rows[base + j, sl]

                acc[i, sl] = tot
            return 0

        lax.fori_loop(0, C, per_node, 0, unroll=False)
        pltpu.sync_copy(acc, out_hbm.at[pl.ds(nb, C)])

    def chunk(t, _):
        cid = w * NCHUNK + t
        nb = cid * C
        pltpu.sync_copy(nei3.at[cid], idx_n)
        pltpu.sync_copy(o3.at[cid], idx_o)
        pltpu.sync_copy(ctx3.at[cid], idx_c)
        hs_n = [pltpu.async_copy(mess.at[idx_n.at[g]],
                                 rows_n.at[pl.ds(g * G, G)], sem_n)
                for g in range(NG)]
        hs_o = [pltpu.async_copy(mess.at[idx_o.at[g]],
                                 rows_o.at[pl.ds(g * G, G)], sem_o)
                for g in range(NG)]
        h_t = pltpu.async_copy(tree.at[idx_c.at[0]], trow, sem_c)
        for h in hs_n:
            h.wait()
        accumulate(rows_n, sumh, nb)
        for h in hs_o:
            h.wait()
        accumulate(rows_o, curo, nb)
        h_t.wait()
        pltpu.sync_copy(trow, tctx.at[pl.ds(nb, C)])
        return 0

    lax.fori_loop(0, NCHUNK, chunk, 0, unroll=False)


@jax.jit
def _sc_gather(mess, nei3, o3, ctx3, tree):
    mesh = plsc.VectorSubcoreMesh(core_axis_name="c", subcore_axis_name="s",
                                  num_cores=NC, num_subcores=NS)
    f = pl.kernel(
        _sc_gather_body,
        out_type=[
            jax.ShapeDtypeStruct((N, HID), jnp.float32),
            jax.ShapeDtypeStruct((N, HID), jnp.float32),
            jax.ShapeDtypeStruct((N, HID), jnp.float32),
        ],
        mesh=mesh,
        scratch_types=[
            pltpu.VMEM((NG, G), jnp.int32),
            pltpu.VMEM((NG, G), jnp.int32),
            pltpu.VMEM((1, C), jnp.int32),
            pltpu.VMEM((C * MAXNB, HID), jnp.float32),
            pltpu.VMEM((C * MAXNB, HID), jnp.float32),
            pltpu.VMEM((C, HID), jnp.float32),
            pltpu.VMEM((C, HID), jnp.float32),
            pltpu.SemaphoreType.DMA,
            pltpu.SemaphoreType.DMA,
            pltpu.SemaphoreType.DMA,
        ],
    )
    return f(mess, nei3, o3, ctx3, tree)


R = 2000


def _tc_body(x_ref, sh_ref, co_ref, tc_ref, pt_ref, st_ref,
             wzx, wzh, bz, wr, br, whx, whh, bh, wuix, wuih, bui,
             wwh, wwt, bw, wwo, bwo, wuh, wut, bu, wuo, buo,
             out_ref):
    f32 = jnp.float32
    xb = x_ref[...]
    sh = sh_ref[...]
    co = co_ref[...]
    tc = tc_ref[...]

    def mm(a, b):
        return jnp.dot(a, b, preferred_element_type=f32)

    z = jax.nn.sigmoid(mm(xb, wzx[...]) + mm(sh, wzh[...]) + bz[...])
    r = jax.nn.sigmoid(mm(xb, wr[...]) + br[...])
    pre = jnp.tanh(mm(xb, whx[...]) + mm(r * sh, whh[...]) + bh[...])
    new_h = (1.0 - z) * sh + z * pre
    stop_hidden = jax.nn.relu(mm(xb, wuix[...]) + mm(co, wuih[...]) + bui[...])
    word_vec = jax.nn.relu(mm(new_h, wwh[...]) + mm(tc, wwt[...]) + bw[...])
    ps = mm(word_vec, wwo[...]) + bwo[...]
    stop_vec = jax.nn.relu(mm(stop_hidden, wuh[...]) + mm(tc, wut[...]) + bu[...])
    ss = mm(stop_vec, wuo[...]) + buo[...]

    m = jnp.max(ps, axis=1, keepdims=True)
    lse = jnp.log(jnp.sum(jnp.exp(ps - m), axis=1, keepdims=True)) + m
    iot = lax.broadcasted_iota(jnp.int32, (R, V), 1)
    tgt = jnp.sum(jnp.where(iot == pt_ref[...], ps, 0.0), axis=1, keepdims=True)
    pred_part = jnp.sum(lse - tgt) * (1.0 / B)

    t = st_ref[...]
    bce = jnp.maximum(ss, 0.0) - ss * t + jnp.log1p(jnp.exp(-jnp.abs(ss)))
    stop_part = jnp.sum(bce) * (1.0 / B)

    part = jnp.concatenate(
        [pred_part.reshape(1, 1), stop_part.reshape(1, 1)], axis=1)

    @pl.when(pl.program_id(0) == 0)
    def _init():
        out_ref[...] = jnp.zeros_like(out_ref)

    out_ref[...] += part


@jax.jit
def _tc_dense(x, sumh, curo, tctx, pt, st, *weights):
    row_spec = lambda width: pl.BlockSpec((R, width), lambda i: (i, 0))
    full = lambda a: pl.BlockSpec(a.shape, lambda i: tuple(0 for _ in a.shape))
    grid_spec = pltpu.PrefetchScalarGridSpec(
        num_scalar_prefetch=0,
        grid=(N // R,),
        in_specs=[
            row_spec(V), row_spec(HID), row_spec(HID), row_spec(HID),
            row_spec(1), row_spec(1),
            *[full(w) for w in weights],
        ],
        out_specs=pl.BlockSpec((1, 2), lambda i: (0, 0)),
    )
    return pl.pallas_call(
        _tc_body,
        grid_spec=grid_spec,
        out_shape=jax.ShapeDtypeStruct((1, 2), jnp.float32),
        compiler_params=pltpu.CompilerParams(
            dimension_semantics=("arbitrary",)),
    )(x, sumh, curo, tctx, pt, st, *weights)


def kernel(x, mess, nei_idx, o_idx, contexts, tree_vecs, pred_targets,
           stop_targets, W_z, b_z, W_r, b_r, W_h, b_h, W_ui, b_ui, W_w, b_w,
           W_wo, b_wo, W_u, b_u, W_uo, b_uo):
    nei3 = nei_idx.reshape(TOTCHUNK, NG, G)
    o3 = o_idx.reshape(TOTCHUNK, NG, G)
    ctx3 = contexts.reshape(TOTCHUNK, 1, C)
    tree_pad = jnp.pad(tree_vecs, ((0, 0), (0, HID - LAT)))
    sumh, curo, tctx = _sc_gather(mess, nei3, o3, ctx3, tree_pad)

    weights = (
        W_z[:V], W_z[V:], b_z.reshape(1, HID),
        W_r, b_r.reshape(1, HID),
        W_h[:V], W_h[V:], b_h.reshape(1, HID),
        W_ui[:V], W_ui[V:], b_ui.reshape(1, HID),
        W_w[:HID], jnp.pad(W_w[HID:], ((0, HID - LAT), (0, 0))), b_w.reshape(1, HID),
        W_wo, b_wo.reshape(1, V),
        W_u[:HID], jnp.pad(W_u[HID:], ((0, HID - LAT), (0, 0))), b_u.reshape(1, HID),
        W_uo, b_uo.reshape(1, 1),
    )
    out = _tc_dense(x, sumh, curo, tctx,
                    pred_targets.reshape(N, 1), stop_targets.reshape(N, 1),
                    *weights)
    return out.reshape(2)


# speedup vs baseline: 3.8379x; 1.1923x over previous
import jax
import jax.numpy as jnp
from jax import lax
from jax.experimental import pallas as pl
from jax.experimental.pallas import tpu as pltpu
from jax.experimental.pallas import tpu_sc as plsc

HID = 128
LAT = 64
V = 6
MAXNB = 10
N = 160000
E = 160000
B = 1024

NC = 2
NS = 16
NW = NC * NS
NPW = N // NW
C = 40
NCHUNK = NPW // C
G = 2 * C
NG = C * MAXNB // G
TOTCHUNK = N // C


def _sc_gather_body(mess, nei3, o3, ctx3, tree,
                    sumh, curo, tctx,
                    idx_n, idx_o, idx_c, rows_n, rows_o, acc, trow,
                    sem_n, sem_o, sem_c):
    w = lax.axis_index("s") * NC + lax.axis_index("c")

    def stage_idx(cid, par):
        pltpu.sync_copy(nei3.at[cid], idx_n.at[par])
        pltpu.sync_copy(o3.at[cid], idx_o.at[par])
        pltpu.sync_copy(ctx3.at[cid], idx_c.at[par])

    def fire_nei(par):
        for g in range(NG):
            pltpu.async_copy(mess.at[idx_n.at[par, g]],
                             rows_n.at[pl.ds(g * G, G)], sem_n)

    def fire_o(par):
        for g in range(NG):
            pltpu.async_copy(mess.at[idx_o.at[par, g]],
                             rows_o.at[pl.ds(g * G, G)], sem_o)

    def fire_tree(par):
        pltpu.async_copy(tree.at[idx_c.at[par, 0]], trow, sem_c)

    def wait_rows(rows, sem):
        for g in range(NG):
            pltpu.make_async_copy(mess.at[idx_n.at[0, g]],
                                  rows.at[pl.ds(g * G, G)], sem).wait()

    def wait_tree():
        pltpu.make_async_copy(tree.at[idx_c.at[0, 0]], trow, sem_c).wait()

    def accumulate(rows, out_hbm, nb):
        def per_node(i, _):
            base = i * MAXNB
            for c8 in range(HID // 16):
                sl = pl.ds(c8 * 16, 16)
                tot = rows[base, sl]
                for j in range(1, MAXNB):
                    tot = tot + rows[base + j, sl]

                acc[i, sl] = tot
            return 0

        lax.fori_loop(0, C, per_node, 0, unroll=False)
        pltpu.sync_copy(acc, out_hbm.at[pl.ds(nb, C)])

    cid0 = w * NCHUNK
    stage_idx(cid0, 0)
    fire_nei(0)
    fire_o(0)
    fire_tree(0)

    def chunk(t, _):
        cur = lax.rem(t, 2)
        nxt = 1 - cur
        cid = w * NCHUNK + t
        nb = cid * C
        stage_idx(cid + 1, nxt)
        wait_rows(rows_n, sem_n)
        accumulate(rows_n, sumh, nb)
        fire_nei(nxt)
        wait_rows(rows_o, sem_o)
        accumulate(rows_o, curo, nb)
        wait_tree()
        pltpu.sync_copy(trow, tctx.at[pl.ds(nb, C)])
        fire_o(nxt)
        fire_tree(nxt)
        return 0

    lax.fori_loop(0, NCHUNK, chunk, 0, unroll=False)

    wait_rows(rows_n, sem_n)
    wait_rows(rows_o, sem_o)
    wait_tree()


@jax.jit
def _sc_gather(mess, nei3, o3, ctx3, tree):
    mesh = plsc.VectorSubcoreMesh(core_axis_name="c", subcore_axis_name="s",
                                  num_cores=NC, num_subcores=NS)
    f = pl.kernel(
        _sc_gather_body,
        out_type=[
            jax.ShapeDtypeStruct((N, HID), jnp.float32),
            jax.ShapeDtypeStruct((N, HID), jnp.float32),
            jax.ShapeDtypeStruct((N, HID), jnp.float32),
        ],
        mesh=mesh,
        scratch_types=[
            pltpu.VMEM((2, NG, G), jnp.int32),
            pltpu.VMEM((2, NG, G), jnp.int32),
            pltpu.VMEM((2, 1, C), jnp.int32),
            pltpu.VMEM((C * MAXNB, HID), jnp.float32),
            pltpu.VMEM((C * MAXNB, HID), jnp.float32),
            pltpu.VMEM((C, HID), jnp.float32),
            pltpu.VMEM((C, HID), jnp.float32),
            pltpu.SemaphoreType.DMA,
            pltpu.SemaphoreType.DMA,
            pltpu.SemaphoreType.DMA,
        ],
    )
    return f(mess, nei3, o3, ctx3, tree)


R = 2000


def _tc_body(x_ref, sh_ref, co_ref, tc_ref, pt_ref, st_ref,
             wzx, wzh, bz, wr, br, whx, whh, bh, wuix, wuih, bui,
             wwh, wwt, bw, wwo, bwo, wuh, wut, bu, wuo, buo,
             out_ref):
    f32 = jnp.float32
    bf16 = jnp.bfloat16
    xb = x_ref[...]
    sh = sh_ref[...]
    co = co_ref[...]
    tc = tc_ref[...]

    def mm(a, b):
        return jnp.dot(a.astype(bf16), b[...], preferred_element_type=f32)

    z = jax.nn.sigmoid(mm(xb, wzx) + mm(sh, wzh) + bz[...])
    r = jax.nn.sigmoid(mm(xb, wr) + br[...])
    pre = jnp.tanh(mm(xb, whx) + mm(r * sh, whh) + bh[...])
    new_h = (1.0 - z) * sh + z * pre
    stop_hidden = jax.nn.relu(mm(xb, wuix) + mm(co, wuih) + bui[...])
    word_vec = jax.nn.relu(mm(new_h, wwh) + mm(tc, wwt) + bw[...])
    ps = mm(word_vec, wwo) + bwo[...]
    stop_vec = jax.nn.relu(mm(stop_hidden, wuh) + mm(tc, wut) + bu[...])
    ss = mm(stop_vec, wuo) + buo[...]

    m = jnp.max(ps, axis=1, keepdims=True)
    lse = jnp.log(jnp.sum(jnp.exp(ps - m), axis=1, keepdims=True)) + m
    iot = lax.broadcasted_iota(jnp.int32, (R, V), 1)
    tgt = jnp.sum(jnp.where(iot == pt_ref[...], ps, 0.0), axis=1, keepdims=True)
    pred_part = jnp.sum(lse - tgt) * (1.0 / B)

    t = st_ref[...]
    bce = jnp.maximum(ss, 0.0) - ss * t + jnp.log1p(jnp.exp(-jnp.abs(ss)))
    stop_part = jnp.sum(bce) * (1.0 / B)

    part = jnp.concatenate(
        [pred_part.reshape(1, 1), stop_part.reshape(1, 1)], axis=1)

    @pl.when(pl.program_id(0) == 0)
    def _init():
        out_ref[...] = jnp.zeros_like(out_ref)

    out_ref[...] += part


@jax.jit
def _tc_dense(x, sumh, curo, tctx, pt, st, *weights):
    row_spec = lambda width: pl.BlockSpec((R, width), lambda i: (i, 0))
    full = lambda a: pl.BlockSpec(a.shape, lambda i: tuple(0 for _ in a.shape))
    grid_spec = pltpu.PrefetchScalarGridSpec(
        num_scalar_prefetch=0,
        grid=(N // R,),
        in_specs=[
            row_spec(V), row_spec(HID), row_spec(HID), row_spec(HID),
            row_spec(1), row_spec(1),
            *[full(w) for w in weights],
        ],
        out_specs=pl.BlockSpec((1, 2), lambda i: (0, 0)),
    )
    return pl.pallas_call(
        _tc_body,
        grid_spec=grid_spec,
        out_shape=jax.ShapeDtypeStruct((1, 2), jnp.float32),
        compiler_params=pltpu.CompilerParams(
            dimension_semantics=("arbitrary",)),
    )(x, sumh, curo, tctx, pt, st, *weights)


def kernel(x, mess, nei_idx, o_idx, contexts, tree_vecs, pred_targets,
           stop_targets, W_z, b_z, W_r, b_r, W_h, b_h, W_ui, b_ui, W_w, b_w,
           W_wo, b_wo, W_u, b_u, W_uo, b_uo):
    nei3 = jnp.pad(nei_idx.reshape(TOTCHUNK, NG, G), ((0, 1), (0, 0), (0, 0)))
    o3 = jnp.pad(o_idx.reshape(TOTCHUNK, NG, G), ((0, 1), (0, 0), (0, 0)))
    ctx3 = jnp.pad(contexts.reshape(TOTCHUNK, 1, C), ((0, 1), (0, 0), (0, 0)))
    tree_pad = jnp.pad(tree_vecs, ((0, 0), (0, HID - LAT)))
    sumh, curo, tctx = _sc_gather(mess, nei3, o3, ctx3, tree_pad)

    bfw = lambda a: a.astype(jnp.bfloat16)
    weights = (
        bfw(W_z[:V]), bfw(W_z[V:]), b_z.reshape(1, HID),
        bfw(W_r), b_r.reshape(1, HID),
        bfw(W_h[:V]), bfw(W_h[V:]), b_h.reshape(1, HID),
        bfw(W_ui[:V]), bfw(W_ui[V:]), b_ui.reshape(1, HID),
        bfw(W_w[:HID]), bfw(jnp.pad(W_w[HID:], ((0, HID - LAT), (0, 0)))),
        b_w.reshape(1, HID),
        bfw(W_wo), b_wo.reshape(1, V),
        bfw(W_u[:HID]), bfw(jnp.pad(W_u[HID:], ((0, HID - LAT), (0, 0)))),
        b_u.reshape(1, HID),
        bfw(W_uo), b_uo.reshape(1, 1),
    )
    out = _tc_dense(x, sumh, curo, tctx,
                    pred_targets.reshape(N, 1), stop_targets.reshape(N, 1),
                    *weights)
    return out.reshape(2)
